# baseline (speedup 1.0000x reference)
import functools

import jax
import jax.numpy as jnp
from jax import lax
from jax.experimental import pallas as pl
from jax.experimental.pallas import tpu as pltpu
from jax.experimental.pallas import tpu_sc as plsc

N_TOKENS = 1000000
N_HIDDEN = 64
BATCH = 16384
N_NEG = 5
TOTAL = BATCH * (1 + N_NEG)
NC = 2
NS = 16
NW = NC * NS
PER_W = TOTAL // NW
CHUNK = 512
N_CHUNKS = PER_W // CHUNK


def _sc_dots_body(u_idx_hbm, v_idx_hbm, u_table_hbm, v_table_hbm, out_hbm,
                  idx_u_v, idx_v_v, u_rows, v_rows, dots_v, sem_u, sem_v):
    wid = lax.axis_index("s") * NC + lax.axis_index("c")
    lane = lax.iota(jnp.int32, 16)

    def chunk_body(c, _):
        base = wid * PER_W + c * CHUNK
        pltpu.sync_copy(u_idx_hbm.at[pl.ds(base, CHUNK)], idx_u_v)
        pltpu.sync_copy(v_idx_hbm.at[pl.ds(base, CHUNK)], idx_v_v)
        cp_u = pltpu.async_copy(u_table_hbm.at[idx_u_v], u_rows, sem_u)
        cp_v = pltpu.async_copy(v_table_hbm.at[idx_v_v], v_rows, sem_v)
        cp_u.wait()
        cp_v.wait()

        def group_body(g, _):
            rows = g * 16 + lane

            def d_body(d, acc):
                cols = jnp.full((16,), d, jnp.int32)
                uu = plsc.load_gather(u_rows, [rows, cols])
                vv = plsc.load_gather(v_rows, [rows, cols])
                return acc + uu * vv

            acc = lax.fori_loop(0, N_HIDDEN, d_body,
                                jnp.zeros((16,), jnp.float32))
            dots_v[pl.ds(g * 16, 16)] = acc
            return 0

        lax.fori_loop(0, CHUNK // 16, group_body, 0)
        pltpu.sync_copy(dots_v, out_hbm.at[pl.ds(base, CHUNK)])
        return 0

    lax.fori_loop(0, N_CHUNKS, chunk_body, 0)


_sc_dots = functools.partial(
    pl.kernel,
    mesh=plsc.VectorSubcoreMesh(core_axis_name="c", subcore_axis_name="s"),
    out_type=jax.ShapeDtypeStruct((TOTAL,), jnp.float32),
    compiler_params=pltpu.CompilerParams(
        needs_layout_passes=False, use_tc_tiling_on_sc=False),
    scratch_types=[
        pltpu.VMEM((CHUNK,), jnp.int32),
        pltpu.VMEM((CHUNK,), jnp.int32),
        pltpu.VMEM((CHUNK, N_HIDDEN), jnp.float32),
        pltpu.VMEM((CHUNK, N_HIDDEN), jnp.float32),
        pltpu.VMEM((CHUNK,), jnp.float32),
        pltpu.SemaphoreType.DMA,
        pltpu.SemaphoreType.DMA,
    ],
)(_sc_dots_body)


_ROWS = TOTAL // 128
_POS_ROWS = BATCH // 128


def _tc_loss_body(dots_ref, out_ref):
    x = dots_ref[...]
    row = lax.broadcasted_iota(jnp.int32, (_ROWS, 128), 0)
    sgn = jnp.where(row < _POS_ROWS, 1.0, -1.0).astype(jnp.float32)
    z = x * sgn
    ls = jnp.minimum(z, 0.0) - jnp.log1p(jnp.exp(-jnp.abs(z)))
    out_ref[0, 0] = -jnp.sum(ls)


def kernel(u_pos, v_pos, u_neg, v_neg, u_table, v_table):
    u_idx = jnp.concatenate(
        [u_pos.astype(jnp.int32), u_neg.reshape(-1).astype(jnp.int32)])
    v_idx = jnp.concatenate(
        [v_pos.astype(jnp.int32), v_neg.reshape(-1).astype(jnp.int32)])
    dots = _sc_dots(u_idx, v_idx, u_table, v_table)
    loss = pl.pallas_call(
        _tc_loss_body,
        out_shape=jax.ShapeDtypeStruct((1, 1), jnp.float32),
        out_specs=pl.BlockSpec(memory_space=pltpu.SMEM),
    )(dots.reshape(_ROWS, 128))
    return loss[0, 0]
